# baseline (speedup 1.0000x reference)
import jax
import jax.numpy as jnp
from jax.experimental import pallas as pl

_COLS = 1024
_NCHUNK = 2


def _gate_kernel(x_ref, w_ref, b_ref, o_ref):
    logits = jax.lax.dot_general(
        x_ref[...], w_ref[...], (((1,), (1,)), ((), ())),
        preferred_element_type=jnp.float32)
    o_ref[...] = jax.nn.sigmoid(logits + b_ref[0][None, :])


def _update_kernel(idx_ref, a_ref, m_ref, k_ref, v_ref, o_ref):
    n0 = pl.program_id(0) * _NCHUNK
    idx = idx_ref[...]
    m = m_ref[...]
    alpha = a_ref[...]
    k = k_ref[...]
    v = v_ref[...]
    slot = n0 + jax.lax.broadcasted_iota(jnp.int32, (1, _NCHUNK), 1)
    active = jnp.any(idx[:, None, :] == slot[:, :, None], axis=2)
    upd = m * jnp.swapaxes(alpha, 2, 3) + jnp.swapaxes(k, 2, 3) * v
    o_ref[...] = jnp.where(active[:, :, None, None], upd, m)


def kernel(M, M_k, M_v, indices_update, x_t, W, b):
    B, N, D, _ = M.shape
    input_dim = x_t.shape[1]
    out_dim = W.shape[0]
    idx = indices_update.astype(jnp.int32)

    alpha = pl.pallas_call(
        _gate_kernel,
        grid=(out_dim // _COLS,),
        in_specs=[
            pl.BlockSpec((B, input_dim), lambda j: (0, 0)),
            pl.BlockSpec((_COLS, input_dim), lambda j: (j, 0)),
            pl.BlockSpec((1, _COLS), lambda j: (0, j)),
        ],
        out_specs=pl.BlockSpec((B, _COLS), lambda j: (0, j)),
        out_shape=jax.ShapeDtypeStruct((B, out_dim), jnp.float32),
    )(x_t, W, b.reshape(1, out_dim))

    alpha4 = alpha.reshape(B, N, 1, D)
    Mk4 = M_k.reshape(B, N, 1, D)
    Mv4 = M_v.reshape(B, N, 1, D)

    return pl.pallas_call(
        _update_kernel,
        grid=(N // _NCHUNK,),
        in_specs=[
            pl.BlockSpec(idx.shape, lambda n: (0, 0)),
            pl.BlockSpec((B, _NCHUNK, 1, D), lambda n: (0, n, 0, 0)),
            pl.BlockSpec((B, _NCHUNK, D, D), lambda n: (0, n, 0, 0)),
            pl.BlockSpec((B, _NCHUNK, 1, D), lambda n: (0, n, 0, 0)),
            pl.BlockSpec((B, _NCHUNK, 1, D), lambda n: (0, n, 0, 0)),
        ],
        out_specs=pl.BlockSpec((B, _NCHUNK, D, D), lambda n: (0, n, 0, 0)),
        out_shape=jax.ShapeDtypeStruct((B, N, D, D), M.dtype),
    )(idx, alpha4, M, Mk4, Mv4)


# speedup vs baseline: 1.4536x; 1.4536x over previous
import jax
import jax.numpy as jnp
from jax.experimental import pallas as pl

_NCHUNK = 2


def _update_kernel(idx_ref, x_ref, w_ref, b_ref, m_ref, k_ref, v_ref, o_ref):
    n0 = pl.program_id(0) * _NCHUNK
    x = x_ref[...]
    idx = idx_ref[...]
    for c in range(_NCHUNK):
        w = w_ref[c]
        logits = jax.lax.dot_general(
            x, w, (((1,), (1,)), ((), ())),
            preferred_element_type=jnp.float32)
        alpha = jax.nn.sigmoid(logits + b_ref[c, 0][None, :])
        active = jnp.any(idx == n0 + c, axis=1)
        m = m_ref[:, c]
        k = k_ref[:, c, 0]
        v = v_ref[:, c, 0]
        upd = m * alpha[:, :, None] + k[:, :, None] * v[:, None, :]
        o_ref[:, c] = jnp.where(active[:, None, None], upd, m)


def kernel(M, M_k, M_v, indices_update, x_t, W, b):
    B, N, D, _ = M.shape
    input_dim = x_t.shape[1]
    idx = indices_update.astype(jnp.int32)
    W3 = W.reshape(N, D, input_dim)
    b3 = b.reshape(N, 1, D)
    Mk4 = M_k.reshape(B, N, 1, D)
    Mv4 = M_v.reshape(B, N, 1, D)

    return pl.pallas_call(
        _update_kernel,
        grid=(N // _NCHUNK,),
        in_specs=[
            pl.BlockSpec(idx.shape, lambda n: (0, 0)),
            pl.BlockSpec((B, input_dim), lambda n: (0, 0)),
            pl.BlockSpec((_NCHUNK, D, input_dim), lambda n: (n, 0, 0)),
            pl.BlockSpec((_NCHUNK, 1, D), lambda n: (n, 0, 0)),
            pl.BlockSpec((B, _NCHUNK, D, D), lambda n: (0, n, 0, 0)),
            pl.BlockSpec((B, _NCHUNK, 1, D), lambda n: (0, n, 0, 0)),
            pl.BlockSpec((B, _NCHUNK, 1, D), lambda n: (0, n, 0, 0)),
        ],
        out_specs=pl.BlockSpec((B, _NCHUNK, D, D), lambda n: (0, n, 0, 0)),
        out_shape=jax.ShapeDtypeStruct((B, N, D, D), M.dtype),
    )(idx, x_t, W3, b3, M, Mk4, Mv4)


# speedup vs baseline: 1.5234x; 1.0480x over previous
import jax
import jax.numpy as jnp
from jax.experimental import pallas as pl

_NCHUNK = 4


def _update_kernel(idx_ref, x_ref, w_ref, b_ref, m_ref, k_ref, v_ref, o_ref):
    n0 = pl.program_id(0) * _NCHUNK
    x = x_ref[...]
    idx = idx_ref[...]
    for c in range(_NCHUNK):
        w = w_ref[c]
        logits = jax.lax.dot_general(
            x, w, (((1,), (1,)), ((), ())),
            preferred_element_type=jnp.float32)
        alpha = jax.nn.sigmoid(logits + b_ref[c, 0][None, :])
        active = jnp.any(idx == n0 + c, axis=1)
        m = m_ref[:, c]
        k = k_ref[:, c, 0]
        v = v_ref[:, c, 0]
        upd = m * alpha[:, :, None] + k[:, :, None] * v[:, None, :]
        o_ref[:, c] = jnp.where(active[:, None, None], upd, m)


def kernel(M, M_k, M_v, indices_update, x_t, W, b):
    B, N, D, _ = M.shape
    input_dim = x_t.shape[1]
    idx = indices_update.astype(jnp.int32)
    W3 = W.reshape(N, D, input_dim)
    b3 = b.reshape(N, 1, D)
    Mk4 = M_k.reshape(B, N, 1, D)
    Mv4 = M_v.reshape(B, N, 1, D)

    return pl.pallas_call(
        _update_kernel,
        grid=(N // _NCHUNK,),
        in_specs=[
            pl.BlockSpec(idx.shape, lambda n: (0, 0)),
            pl.BlockSpec((B, input_dim), lambda n: (0, 0)),
            pl.BlockSpec((_NCHUNK, D, input_dim), lambda n: (n, 0, 0)),
            pl.BlockSpec((_NCHUNK, 1, D), lambda n: (n, 0, 0)),
            pl.BlockSpec((B, _NCHUNK, D, D), lambda n: (0, n, 0, 0)),
            pl.BlockSpec((B, _NCHUNK, 1, D), lambda n: (0, n, 0, 0)),
            pl.BlockSpec((B, _NCHUNK, 1, D), lambda n: (0, n, 0, 0)),
        ],
        out_specs=pl.BlockSpec((B, _NCHUNK, D, D), lambda n: (0, n, 0, 0)),
        out_shape=jax.ShapeDtypeStruct((B, N, D, D), M.dtype),
    )(idx, x_t, W3, b3, M, Mk4, Mv4)
